# baseline (speedup 1.0000x reference)
import functools

import jax
import jax.numpy as jnp
from jax import lax
from jax.experimental import pallas as pl
from jax.experimental.pallas import tpu as pltpu
from jax.experimental.pallas import tpu_sc as plsc

_N_NODES = 10000
_N_EDGES = 320000
_NC = 1
_NS = 16
_NW = _NC * _NS
_L = 16

_NBLK = _N_EDGES // 128
_BSML = _NBLK // _NW
_BBIG = _BSML + 1
_NBIG = _NBLK % _NW
_WBIG = _BBIG * 128
_NCHUNK = 4

_EPW = _N_EDGES // _NW


def _chunks(n):
    q, r = divmod(n, _NCHUNK)
    parts = [q + 1] * r + [q] * (_NCHUNK - r)
    offs, o = [], 0
    for p in parts:
        offs.append(o)
        o += p
    return list(zip(offs, parts))


def _rowsum_body(h_ref, o_ref):
    o_ref[...] = jnp.sum(h_ref[...], axis=1)


def _rowsum(h):
    return pl.pallas_call(
        _rowsum_body,
        out_shape=jax.ShapeDtypeStruct((_N_NODES,), jnp.float32),
    )(h)


_sc_mesh = plsc.VectorSubcoreMesh(
    core_axis_name="c", subcore_axis_name="s", num_cores=_NC)


@functools.partial(
    pl.kernel,
    out_type=jax.ShapeDtypeStruct((_N_EDGES,), jnp.int32),
    mesh=_sc_mesh,
    compiler_params=pltpu.CompilerParams(needs_layout_passes=False),
    scratch_types=[
        pltpu.VMEM((2, _WBIG), jnp.int32),
        pltpu.SemaphoreType.DMA,
        pltpu.SemaphoreType.DMA,
    ],
)
def _extract_kernel(edge_hbm, src_hbm, edges_v, sem_e, sem_o):
    wid = lax.axis_index("s") * _NC + lax.axis_index("c")
    big = wid < _NBIG
    base = 128 * jnp.where(big, wid * _BBIG, _NBIG + wid * _BSML)

    def run(nblocks):
        in_cps = []
        for off, nb in _chunks(nblocks):
            in_cps.append(pltpu.async_copy(
                edge_hbm.at[:, pl.ds(base + off * 128, nb * 128)],
                edges_v.at[:, pl.ds(off * 128, nb * 128)], sem_e))
        out_cps = []
        for cp, (off, nb) in zip(in_cps, _chunks(nblocks)):
            cp.wait()
            out_cps.append(pltpu.async_copy(
                edges_v.at[0, pl.ds(off * 128, nb * 128)],
                src_hbm.at[pl.ds(base + off * 128, nb * 128)], sem_o))
        for cp in out_cps:
            cp.wait()

    @pl.when(big)
    def _():
        run(_BBIG)

    @pl.when(jnp.logical_not(big))
    def _():
        run(_BSML)


@functools.partial(
    pl.kernel,
    out_type=jax.ShapeDtypeStruct((_N_EDGES,), jnp.float32),
    mesh=_sc_mesh,
    compiler_params=pltpu.CompilerParams(needs_layout_passes=False),
    scratch_types=[
        pltpu.VMEM((_N_NODES,), jnp.float32),
        pltpu.VMEM((_EPW,), jnp.int32),
        pltpu.VMEM((_EPW,), jnp.float32),
        pltpu.SemaphoreType.DMA,
        pltpu.SemaphoreType.DMA,
        pltpu.SemaphoreType.DMA,
    ],
)
def _gather_kernel(s_hbm, src_hbm, out_hbm, table_v, idx_v, out_v,
                   sem_t, sem_e, sem_o):
    wid = lax.axis_index("s") * _NC + lax.axis_index("c")
    base = wid * _EPW
    tbl_cp = pltpu.async_copy(s_hbm, table_v, sem_t)
    groups = _chunks(_EPW // _L)
    in_cps = []
    for goff, gn in groups:
        in_cps.append(pltpu.async_copy(
            src_hbm.at[pl.ds(base + goff * _L, gn * _L)],
            idx_v.at[pl.ds(goff * _L, gn * _L)], sem_e))
    tbl_cp.wait()
    out_cps = []
    for cp, (goff, gn) in zip(in_cps, groups):
        cp.wait()

        @plsc.parallel_loop(goff, goff + gn, unroll=8)
        def body(g):
            idxs = idx_v[pl.ds(g * _L, _L)]
            out_v[pl.ds(g * _L, _L)] = plsc.load_gather(table_v, [idxs])

        out_cps.append(pltpu.async_copy(
            out_v.at[pl.ds(goff * _L, gn * _L)],
            out_hbm.at[pl.ds(base + goff * _L, gn * _L)], sem_o))
    for cp in out_cps:
        cp.wait()


def kernel(h, W, b, edge_index):
    src = _extract_kernel(edge_index.astype(jnp.int32))
    s = _rowsum(h)
    return _gather_kernel(s, src)


# speedup vs baseline: 1.0572x; 1.0572x over previous
import functools

import jax
import jax.numpy as jnp
from jax import lax
from jax.experimental import pallas as pl
from jax.experimental.pallas import tpu as pltpu
from jax.experimental.pallas import tpu_sc as plsc

_N_NODES = 10000
_N_EDGES = 320000
_NC = 1
_NS = 16
_NW = _NC * _NS
_L = 16

_NBLK = _N_EDGES // 128
_BSML = _NBLK // _NW
_BBIG = _BSML + 1
_NBIG = _NBLK % _NW
_WBIG = _BBIG * 128
_NCHUNK = 6


def _chunks(nblocks):
    q, r = divmod(nblocks, _NCHUNK)
    parts = [q + 1] * r + [q] * (_NCHUNK - r)
    offs, o = [], 0
    for p in parts:
        offs.append(o)
        o += p
    return list(zip(offs, parts))


def _rowsum_body(h_ref, o_ref):
    o_ref[...] = jnp.sum(h_ref[...], axis=1)


def _rowsum(h):
    return pl.pallas_call(
        _rowsum_body,
        out_shape=jax.ShapeDtypeStruct((_N_NODES,), jnp.float32),
    )(h)


_gather_mesh = plsc.VectorSubcoreMesh(
    core_axis_name="c", subcore_axis_name="s", num_cores=_NC)


@functools.partial(
    pl.kernel,
    out_type=jax.ShapeDtypeStruct((_N_EDGES,), jnp.float32),
    mesh=_gather_mesh,
    compiler_params=pltpu.CompilerParams(needs_layout_passes=False),
    scratch_types=[
        pltpu.VMEM((_N_NODES,), jnp.float32),
        pltpu.VMEM((2, _WBIG), jnp.int32),
        pltpu.VMEM((_WBIG,), jnp.float32),
        pltpu.SemaphoreType.DMA,
        pltpu.SemaphoreType.DMA,
        pltpu.SemaphoreType.DMA,
    ],
)
def _gather_kernel(s_hbm, edge_hbm, out_hbm, table_v, edges_v, out_v,
                   sem_t, sem_e, sem_o):
    wid = lax.axis_index("s") * _NC + lax.axis_index("c")
    big = wid < _NBIG
    base = 128 * jnp.where(big, wid * _BBIG, _NBIG + wid * _BSML)
    tbl_cp = pltpu.async_copy(s_hbm, table_v, sem_t)

    def run(nblocks):
        in_cps = []
        for off, nb in _chunks(nblocks):
            in_cps.append(pltpu.async_copy(
                edge_hbm.at[:, pl.ds(base + off * 128, nb * 128)],
                edges_v.at[:, pl.ds(off * 128, nb * 128)], sem_e))
        tbl_cp.wait()
        out_cps = []
        for cp, (off, nb) in zip(in_cps, _chunks(nblocks)):
            cp.wait()

            @plsc.parallel_loop(off * 8, (off + nb) * 8, unroll=8)
            def body(g):
                idxs = edges_v[0, pl.ds(g * _L, _L)]
                out_v[pl.ds(g * _L, _L)] = plsc.load_gather(table_v, [idxs])

            out_cps.append(pltpu.async_copy(
                out_v.at[pl.ds(off * 128, nb * 128)],
                out_hbm.at[pl.ds(base + off * 128, nb * 128)], sem_o))
        for cp in out_cps:
            cp.wait()

    @pl.when(big)
    def _():
        run(_BBIG)

    @pl.when(jnp.logical_not(big))
    def _():
        run(_BSML)


def kernel(h, W, b, edge_index):
    s = _rowsum(h)
    return _gather_kernel(s, edge_index.astype(jnp.int32))


# speedup vs baseline: 1.0708x; 1.0128x over previous
import functools

import jax
import jax.numpy as jnp
from jax import lax
from jax.experimental import pallas as pl
from jax.experimental.pallas import tpu as pltpu
from jax.experimental.pallas import tpu_sc as plsc

_N_NODES = 10000
_N_EDGES = 320000
_NC = 1
_NS = 16
_NW = _NC * _NS
_L = 16

_NBLK = _N_EDGES // 128
_BSML = _NBLK // _NW
_BBIG = _BSML + 1
_NBIG = _NBLK % _NW
_WBIG = _BBIG * 128
_NCHUNK = 6


def _chunks(nblocks):
    q, r = divmod(nblocks, _NCHUNK)
    parts = [q + 1] * r + [q] * (_NCHUNK - r)
    offs, o = [], 0
    for p in parts:
        offs.append(o)
        o += p
    return list(zip(offs, parts))


def _rowsum_body(h_ref, o_ref):
    o_ref[...] = jnp.sum(h_ref[...], axis=1)


def _rowsum(h):
    return pl.pallas_call(
        _rowsum_body,
        out_shape=jax.ShapeDtypeStruct((_N_NODES,), jnp.float32),
    )(h)


_gather_mesh = plsc.VectorSubcoreMesh(
    core_axis_name="c", subcore_axis_name="s", num_cores=_NC)


@functools.partial(
    pl.kernel,
    out_type=jax.ShapeDtypeStruct((_N_EDGES,), jnp.float32),
    mesh=_gather_mesh,
    compiler_params=pltpu.CompilerParams(needs_layout_passes=False),
    scratch_types=[
        pltpu.VMEM((_N_NODES,), jnp.float32),
        pltpu.VMEM((2, _WBIG), jnp.int32),
        pltpu.VMEM((_WBIG,), jnp.float32),
        pltpu.SemaphoreType.DMA,
        pltpu.SemaphoreType.DMA,
        pltpu.SemaphoreType.DMA,
    ],
)
def _gather_kernel(s_hbm, edge_hbm, out_hbm, table_v, edges_v, out_v,
                   sem_t, sem_e, sem_o):
    wid = lax.axis_index("s") * _NC + lax.axis_index("c")
    base = 128 * jnp.minimum(wid * _BBIG, _NBLK - _BBIG)
    tbl_cp = pltpu.async_copy(s_hbm, table_v, sem_t)

    in_cps = []
    for off, nb in _chunks(_BBIG):
        in_cps.append(pltpu.async_copy(
            edge_hbm.at[:, pl.ds(base + off * 128, nb * 128)],
            edges_v.at[:, pl.ds(off * 128, nb * 128)], sem_e))
    tbl_cp.wait()
    out_cps = []
    for cp, (off, nb) in zip(in_cps, _chunks(_BBIG)):
        cp.wait()

        @plsc.parallel_loop(off * 8, (off + nb) * 8, unroll=8)
        def body(g):
            idxs = edges_v[0, pl.ds(g * _L, _L)]
            out_v[pl.ds(g * _L, _L)] = plsc.load_gather(table_v, [idxs])

        out_cps.append(pltpu.async_copy(
            out_v.at[pl.ds(off * 128, nb * 128)],
            out_hbm.at[pl.ds(base + off * 128, nb * 128)], sem_o))
    for cp in out_cps:
        cp.wait()


def kernel(h, W, b, edge_index):
    s = _rowsum(h)
    return _gather_kernel(s, edge_index.astype(jnp.int32))
